# Initial kernel scaffold; baseline (speedup 1.0000x reference)
import jax
import jax.numpy as jnp
from jax.experimental import pallas as pl


def kernel(x, edge_index, edge_weight, W1, b1, g1, be1, W2, b2, g2, be2, Wl1, bl1, Wl2, bl2):
    raise NotImplementedError("write your pallas kernel here")


# speedup vs baseline: 2.1539x; 2.1539x over previous
import functools

import jax
import jax.numpy as jnp
from jax.experimental import pallas as pl
from jax.experimental.pallas import tpu as pltpu

_EPS = 1e-5
_BN = 1000


def _mm_stats_body(a_ref, w_ref, b_ref, h_ref, st_ref):
    i = pl.program_id(0)
    h = jnp.dot(a_ref[...], w_ref[...], preferred_element_type=jnp.float32)
    h = h + b_ref[...]
    h_ref[...] = h

    @pl.when(i == 0)
    def _():
        st_ref[...] = jnp.zeros_like(st_ref)

    st_ref[0, :] += jnp.sum(h, axis=0)
    st_ref[1, :] += jnp.sum(h * h, axis=0)


def _mm_stats(a, w, b):
    n, k = a.shape
    m = w.shape[1]
    grid = n // _BN
    return pl.pallas_call(
        _mm_stats_body,
        grid=(grid,),
        in_specs=[
            pl.BlockSpec((_BN, k), lambda i: (i, 0)),
            pl.BlockSpec((k, m), lambda i: (0, 0)),
            pl.BlockSpec((1, m), lambda i: (0, 0)),
        ],
        out_specs=[
            pl.BlockSpec((_BN, m), lambda i: (i, 0)),
            pl.BlockSpec((2, m), lambda i: (0, 0)),
        ],
        out_shape=[
            jax.ShapeDtypeStruct((n, m), jnp.float32),
            jax.ShapeDtypeStruct((2, m), jnp.float32),
        ],
    )(a, w.reshape(k, m), b.reshape(1, m))


def _bn_mm_body(h_ref, st_ref, g_ref, be_ref, w_ref, dis_ref, hs_ref, *, n):
    mu = st_ref[0, :] * (1.0 / n)
    var = st_ref[1, :] * (1.0 / n) - mu * mu
    scale = jax.lax.rsqrt(var + _EPS) * g_ref[0, :]
    hn = jnp.maximum((h_ref[...] - mu[None, :]) * scale[None, :]
                     + be_ref[0, :][None, :], 0.0)
    hw = jnp.dot(hn, w_ref[...], preferred_element_type=jnp.float32)
    hs_ref[...] = hw * dis_ref[...]


def _bn_mm(h, stats, g, be, w, dis):
    n, k = h.shape
    m = w.shape[1]
    grid = n // _BN
    return pl.pallas_call(
        functools.partial(_bn_mm_body, n=n),
        grid=(grid,),
        in_specs=[
            pl.BlockSpec((_BN, k), lambda i: (i, 0)),
            pl.BlockSpec((2, k), lambda i: (0, 0)),
            pl.BlockSpec((1, k), lambda i: (0, 0)),
            pl.BlockSpec((1, k), lambda i: (0, 0)),
            pl.BlockSpec((k, m), lambda i: (0, 0)),
            pl.BlockSpec((_BN, 1), lambda i: (i, 0)),
        ],
        out_specs=pl.BlockSpec((_BN, m), lambda i: (i, 0)),
        out_shape=jax.ShapeDtypeStruct((n, m), jnp.float32),
    )(h, stats, g.reshape(1, k), be.reshape(1, k), w, dis)


def _comb_stats_body(agg_ref, hs_ref, dis_ref, b_ref, h_ref, st_ref):
    i = pl.program_id(0)
    h = (agg_ref[...] + hs_ref[...]) * dis_ref[...] + b_ref[...]
    h_ref[...] = h

    @pl.when(i == 0)
    def _():
        st_ref[...] = jnp.zeros_like(st_ref)

    st_ref[0, :] += jnp.sum(h, axis=0)
    st_ref[1, :] += jnp.sum(h * h, axis=0)


def _comb_stats(agg, hs, dis, b):
    n, m = hs.shape
    grid = n // _BN
    return pl.pallas_call(
        _comb_stats_body,
        grid=(grid,),
        in_specs=[
            pl.BlockSpec((_BN, m), lambda i: (i, 0)),
            pl.BlockSpec((_BN, m), lambda i: (i, 0)),
            pl.BlockSpec((_BN, 1), lambda i: (i, 0)),
            pl.BlockSpec((1, m), lambda i: (0, 0)),
        ],
        out_specs=[
            pl.BlockSpec((_BN, m), lambda i: (i, 0)),
            pl.BlockSpec((2, m), lambda i: (0, 0)),
        ],
        out_shape=[
            jax.ShapeDtypeStruct((n, m), jnp.float32),
            jax.ShapeDtypeStruct((2, m), jnp.float32),
        ],
    )(agg, hs, dis, b.reshape(1, m))


def _head_body(h_ref, st_ref, g_ref, be_ref, w1_ref, b1_ref, w2_ref, b2_ref,
               o_ref, *, n, d_out):
    mu = st_ref[0, :] * (1.0 / n)
    var = st_ref[1, :] * (1.0 / n) - mu * mu
    scale = jax.lax.rsqrt(var + _EPS) * g_ref[0, :]
    hn = jnp.maximum((h_ref[...] - mu[None, :]) * scale[None, :]
                     + be_ref[0, :][None, :], 0.0)
    m1 = jnp.maximum(jnp.dot(hn, w1_ref[...],
                             preferred_element_type=jnp.float32)
                     + b1_ref[...], 0.0)
    o = jnp.dot(m1, w2_ref[...], preferred_element_type=jnp.float32) + b2_ref[...]
    pad = o.shape[1]
    lane = jax.lax.broadcasted_iota(jnp.int32, o.shape, 1)
    valid = lane < d_out
    om = jnp.where(valid, o, -jnp.inf)
    mx = jnp.max(om, axis=1, keepdims=True)
    ex = jnp.where(valid, jnp.exp(om - mx), 0.0)
    lse = jnp.log(jnp.sum(ex, axis=1, keepdims=True)) + mx
    o_ref[...] = o - lse


def _head(h, stats, g, be, wl1, bl1, wl2, bl2, d_out):
    n, k = h.shape
    dm = wl1.shape[1]
    pad = 128
    wl2p = jnp.zeros((dm, pad), jnp.float32).at[:, :d_out].set(wl2)
    bl2p = jnp.zeros((1, pad), jnp.float32).at[0, :d_out].set(bl2)
    grid = n // _BN
    out = pl.pallas_call(
        functools.partial(_head_body, n=n, d_out=d_out),
        grid=(grid,),
        in_specs=[
            pl.BlockSpec((_BN, k), lambda i: (i, 0)),
            pl.BlockSpec((2, k), lambda i: (0, 0)),
            pl.BlockSpec((1, k), lambda i: (0, 0)),
            pl.BlockSpec((1, k), lambda i: (0, 0)),
            pl.BlockSpec((k, dm), lambda i: (0, 0)),
            pl.BlockSpec((1, dm), lambda i: (0, 0)),
            pl.BlockSpec((dm, pad), lambda i: (0, 0)),
            pl.BlockSpec((1, pad), lambda i: (0, 0)),
        ],
        out_specs=pl.BlockSpec((_BN, pad), lambda i: (i, 0)),
        out_shape=jax.ShapeDtypeStruct((n, pad), jnp.float32),
    )(h, stats, g.reshape(1, k), be.reshape(1, k), wl1, bl1.reshape(1, dm),
      wl2p, bl2p)
    return out[:, :d_out]


def _aggregate(hs, row, col, ew):
    msg = hs[row] * ew[:, None]
    return jnp.zeros_like(hs).at[col].add(msg)


def kernel(x, edge_index, edge_weight, W1, b1, g1, be1, W2, b2, g2, be2,
           Wl1, bl1, Wl2, bl2):
    n, d_in = x.shape
    row, col = edge_index[0], edge_index[1]

    deg = jnp.ones((n,), jnp.float32).at[col].add(edge_weight)
    dis = jax.lax.rsqrt(deg)
    dis2 = dis[:, None]

    xs = x * dis2
    agg1 = _aggregate(xs, row, col, edge_weight)
    ax = (agg1 + xs) * dis2

    h1, st1 = _mm_stats(ax, W1, b1)
    hs2 = _bn_mm(h1, st1, g1, be1, W2, dis2)
    agg2 = _aggregate(hs2, row, col, edge_weight)
    h2, st2 = _comb_stats(agg2, hs2, dis2, b2)
    return _head(h2, st2, g2, be2, Wl1, bl1, Wl2, bl2, Wl2.shape[1])


# speedup vs baseline: 5.5243x; 2.5648x over previous
import functools

import jax
import jax.numpy as jnp
from jax import lax
from jax.experimental import pallas as pl
from jax.experimental.pallas import tpu as pltpu
from jax.experimental.pallas import tpu_sc as plsc

_EPS = 1e-5
_BN = 1000
_NW = 32
_NSUB = 16
_B = 80
_ACC_PAD = 10240


def _deg_body(col_hbm, ew_hbm, out_hbm, cbuf, ebuf, dbuf, *, n, e):
    epw = e // _NW
    w = lax.axis_index("s") * 2 + lax.axis_index("c")
    base = pl.multiple_of(w * epw, 8)
    pltpu.sync_copy(col_hbm.at[pl.ds(base, epw)], cbuf)
    pltpu.sync_copy(ew_hbm.at[pl.ds(base, epw)], ebuf)

    zv = jnp.zeros((16,), jnp.float32)

    def zero(k, _):
        dbuf[pl.ds(k * 16, 16)] = zv
        return ()
    lax.fori_loop(0, n // 16, zero, ())

    def body(k, _):
        cv = cbuf[pl.ds(k * 16, 16)]
        ev = ebuf[pl.ds(k * 16, 16)]
        plsc.addupdate_scatter(dbuf, [cv], ev)
        return ()
    lax.fori_loop(0, epw // 16, body, ())
    pltpu.sync_copy(dbuf, out_hbm.at[w])


def _deg_partials(col, ew, n, e):
    epw = e // _NW
    mesh = plsc.VectorSubcoreMesh(core_axis_name="c", subcore_axis_name="s")
    return pl.kernel(
        functools.partial(_deg_body, n=n, e=e),
        out_type=jax.ShapeDtypeStruct((_NW, n), jnp.float32),
        mesh=mesh,
        compiler_params=pltpu.CompilerParams(needs_layout_passes=False),
        scratch_types=[
            pltpu.VMEM((epw,), jnp.int32),
            pltpu.VMEM((epw,), jnp.float32),
            pltpu.VMEM((n,), jnp.float32),
        ],
    )(col, ew)


def _agg_body(tab_hbm, row_hbm, col_hbm, ew_hbm, out_hbm,
              rbuf, cbuf, ebuf, radj, cblk, gbuf, acc, sem,
              *, n, e, nch):
    epw = e // _NW
    nblk = epw // _B
    core = lax.axis_index("c")
    s = lax.axis_index("s")
    w = s * 2 + core
    ebase = pl.multiple_of(w * epw, 8)

    pltpu.sync_copy(row_hbm.at[pl.ds(ebase, epw)], rbuf)
    pltpu.sync_copy(col_hbm.at[pl.ds(ebase, epw)], cbuf)
    pltpu.sync_copy(ew_hbm.at[pl.ds(ebase, epw)], ebuf)

    zv = jnp.zeros((16,), jnp.float32)
    my0 = 640 * s
    last = n - 640 * (_NSUB - 1)

    for ch in range(nch):
        def zrow(i, _):
            for j in range(8):
                gbuf[i, pl.ds(j * 16, 16)] = zv
            return ()
        lax.fori_loop(0, _B, zrow, ())

        @pl.when(s != _NSUB - 1)
        def _():
            for j in range(8):
                pltpu.sync_copy(gbuf, acc.at[pl.ds(my0 + j * _B, _B)])

        @pl.when(s == _NSUB - 1)
        def _():
            for j in range(last // _B):
                pltpu.sync_copy(gbuf, acc.at[pl.ds(my0 + j * _B, _B)])
        plsc.subcore_barrier()

        def blk(b, _):
            off = b * _B
            for k in range(_B // 16):
                rv = rbuf[pl.ds(off + k * 16, 16)]
                radj[pl.ds(k * 16, 16)] = rv * nch + ch
                cblk[pl.ds(k * 16, 16)] = cbuf[pl.ds(off + k * 16, 16)]
            pltpu.async_copy(tab_hbm.at[radj], gbuf, sem).wait()

            def scale(i, _):
                sp = plsc.load_gather(ebuf, [jnp.full((16,), off + i,
                                                      jnp.int32)])
                for j in range(8):
                    gbuf[i, pl.ds(j * 16, 16)] = (
                        gbuf[i, pl.ds(j * 16, 16)] * sp)
                return ()
            lax.fori_loop(0, _B, scale, ())
            pltpu.sync_copy(gbuf, acc.at[cblk], add=True)
            return ()
        lax.fori_loop(0, nblk, blk, ())
        plsc.subcore_barrier()

        @pl.when(s != _NSUB - 1)
        def _():
            pltpu.sync_copy(acc.at[pl.ds(my0, 640)],
                            out_hbm.at[core, ch, pl.ds(my0, 640)])

        @pl.when(s == _NSUB - 1)
        def _():
            pltpu.sync_copy(acc.at[pl.ds(my0, last)],
                            out_hbm.at[core, ch, pl.ds(my0, last)])


def _sc_aggregate(table, row, col, ew, n, e, nch):
    epw = e // _NW
    mesh = plsc.VectorSubcoreMesh(core_axis_name="c", subcore_axis_name="s")
    return pl.kernel(
        functools.partial(_agg_body, n=n, e=e, nch=nch),
        out_type=jax.ShapeDtypeStruct((2, nch, n, 128), jnp.float32),
        mesh=mesh,
        compiler_params=pltpu.CompilerParams(needs_layout_passes=False),
        scratch_types=[
            pltpu.VMEM((epw,), jnp.int32),
            pltpu.VMEM((epw,), jnp.int32),
            pltpu.VMEM((epw,), jnp.float32),
            pltpu.VMEM((_B,), jnp.int32),
            pltpu.VMEM((_B,), jnp.int32),
            pltpu.VMEM((_B, 128), jnp.float32),
            pltpu.VMEM_SHARED((10000, 128), jnp.float32),
            pltpu.SemaphoreType.DMA,
        ],
    )(table, row, col, ew)


def _prep_body(degp_ref, x_ref, dis_ref, xs_ref, *, n):
    d = jnp.sum(degp_ref[...], axis=0, keepdims=True) + 1.0
    t = jnp.transpose(jax.lax.rsqrt(d), (1, 0))
    dis_ref[...] = t
    xs_ref[...] = x_ref[...] * t


def _prep(degp, x):
    n, k = x.shape
    return pl.pallas_call(
        functools.partial(_prep_body, n=n),
        grid=(1,),
        in_specs=[
            pl.BlockSpec((_NW, n), lambda i: (0, 0)),
            pl.BlockSpec((n, k), lambda i: (0, 0)),
        ],
        out_specs=[
            pl.BlockSpec((n, 1), lambda i: (0, 0)),
            pl.BlockSpec((n, k), lambda i: (0, 0)),
        ],
        out_shape=[
            jax.ShapeDtypeStruct((n, 1), jnp.float32),
            jax.ShapeDtypeStruct((n, k), jnp.float32),
        ],
    )(degp, x)


def _mm_stats_body(aggp_ref, xs_ref, dis_ref, w_ref, b_ref, h_ref, st_ref):
    i = pl.program_id(0)
    a = (aggp_ref[0] + aggp_ref[1] + xs_ref[...]) * dis_ref[...]
    h = jnp.dot(a, w_ref[...], preferred_element_type=jnp.float32)
    h = h + b_ref[...]
    h_ref[...] = h

    @pl.when(i == 0)
    def _():
        st_ref[...] = jnp.zeros_like(st_ref)

    st_ref[0, :] += jnp.sum(h, axis=0)
    st_ref[1, :] += jnp.sum(h * h, axis=0)


def _mm_stats(aggp, xs, dis, w, b):
    n, k = xs.shape
    m = w.shape[1]
    return pl.pallas_call(
        _mm_stats_body,
        grid=(n // _BN,),
        in_specs=[
            pl.BlockSpec((2, _BN, k), lambda i: (0, i, 0)),
            pl.BlockSpec((_BN, k), lambda i: (i, 0)),
            pl.BlockSpec((_BN, 1), lambda i: (i, 0)),
            pl.BlockSpec((k, m), lambda i: (0, 0)),
            pl.BlockSpec((1, m), lambda i: (0, 0)),
        ],
        out_specs=[
            pl.BlockSpec((_BN, m), lambda i: (i, 0)),
            pl.BlockSpec((2, m), lambda i: (0, 0)),
        ],
        out_shape=[
            jax.ShapeDtypeStruct((n, m), jnp.float32),
            jax.ShapeDtypeStruct((2, m), jnp.float32),
        ],
    )(aggp, xs, dis, w.reshape(k, m), b.reshape(1, m))


def _bn_mm_body(h_ref, st_ref, g_ref, be_ref, w_ref, dis_ref, hs_ref, *, n):
    mu = st_ref[0, :] * (1.0 / n)
    var = st_ref[1, :] * (1.0 / n) - mu * mu
    scale = jax.lax.rsqrt(var + _EPS) * g_ref[0, :]
    hn = jnp.maximum((h_ref[...] - mu[None, :]) * scale[None, :]
                     + be_ref[0, :][None, :], 0.0)
    hw = jnp.dot(hn, w_ref[...], preferred_element_type=jnp.float32)
    hs_ref[...] = hw * dis_ref[...]


def _bn_mm(h, stats, g, be, w, dis):
    n, k = h.shape
    m = w.shape[1]
    return pl.pallas_call(
        functools.partial(_bn_mm_body, n=n),
        grid=(n // _BN,),
        in_specs=[
            pl.BlockSpec((_BN, k), lambda i: (i, 0)),
            pl.BlockSpec((2, k), lambda i: (0, 0)),
            pl.BlockSpec((1, k), lambda i: (0, 0)),
            pl.BlockSpec((1, k), lambda i: (0, 0)),
            pl.BlockSpec((k, m), lambda i: (0, 0)),
            pl.BlockSpec((_BN, 1), lambda i: (i, 0)),
        ],
        out_specs=pl.BlockSpec((_BN, m), lambda i: (i, 0)),
        out_shape=jax.ShapeDtypeStruct((n, m), jnp.float32),
    )(h, stats, g.reshape(1, k), be.reshape(1, k), w, dis)


def _comb_stats_body(aggp_ref, hs_ref, dis_ref, b_ref, h_ref, st_ref):
    i = pl.program_id(0)
    agg = jnp.concatenate(
        [aggp_ref[c] + aggp_ref[8 + c] for c in range(8)], axis=1)
    h = (agg + hs_ref[...]) * dis_ref[...] + b_ref[...]
    h_ref[...] = h

    @pl.when(i == 0)
    def _():
        st_ref[...] = jnp.zeros_like(st_ref)

    st_ref[0, :] += jnp.sum(h, axis=0)
    st_ref[1, :] += jnp.sum(h * h, axis=0)


def _comb_stats(aggp, hs, dis, b):
    n, m = hs.shape
    return pl.pallas_call(
        _comb_stats_body,
        grid=(n // _BN,),
        in_specs=[
            pl.BlockSpec((16, _BN, 128), lambda i: (0, i, 0)),
            pl.BlockSpec((_BN, m), lambda i: (i, 0)),
            pl.BlockSpec((_BN, 1), lambda i: (i, 0)),
            pl.BlockSpec((1, m), lambda i: (0, 0)),
        ],
        out_specs=[
            pl.BlockSpec((_BN, m), lambda i: (i, 0)),
            pl.BlockSpec((2, m), lambda i: (0, 0)),
        ],
        out_shape=[
            jax.ShapeDtypeStruct((n, m), jnp.float32),
            jax.ShapeDtypeStruct((2, m), jnp.float32),
        ],
    )(aggp, hs, dis, b.reshape(1, m))


def _head_body(h_ref, st_ref, g_ref, be_ref, w1_ref, b1_ref, w2_ref, b2_ref,
               o_ref, *, n, d_out):
    mu = st_ref[0, :] * (1.0 / n)
    var = st_ref[1, :] * (1.0 / n) - mu * mu
    scale = jax.lax.rsqrt(var + _EPS) * g_ref[0, :]
    hn = jnp.maximum((h_ref[...] - mu[None, :]) * scale[None, :]
                     + be_ref[0, :][None, :], 0.0)
    m1 = jnp.maximum(jnp.dot(hn, w1_ref[...],
                             preferred_element_type=jnp.float32)
                     + b1_ref[...], 0.0)
    o = jnp.dot(m1, w2_ref[...], preferred_element_type=jnp.float32) + b2_ref[...]
    lane = jax.lax.broadcasted_iota(jnp.int32, o.shape, 1)
    valid = lane < d_out
    om = jnp.where(valid, o, -jnp.inf)
    mx = jnp.max(om, axis=1, keepdims=True)
    ex = jnp.where(valid, jnp.exp(om - mx), 0.0)
    lse = jnp.log(jnp.sum(ex, axis=1, keepdims=True)) + mx
    o_ref[...] = o - lse


def _head(h, stats, g, be, wl1, bl1, wl2, bl2, d_out):
    n, k = h.shape
    dm = wl1.shape[1]
    pad = 128
    wl2p = jnp.zeros((dm, pad), jnp.float32).at[:, :d_out].set(wl2)
    bl2p = jnp.zeros((1, pad), jnp.float32).at[0, :d_out].set(bl2)
    out = pl.pallas_call(
        functools.partial(_head_body, n=n, d_out=d_out),
        grid=(n // _BN,),
        in_specs=[
            pl.BlockSpec((_BN, k), lambda i: (i, 0)),
            pl.BlockSpec((2, k), lambda i: (0, 0)),
            pl.BlockSpec((1, k), lambda i: (0, 0)),
            pl.BlockSpec((1, k), lambda i: (0, 0)),
            pl.BlockSpec((k, dm), lambda i: (0, 0)),
            pl.BlockSpec((1, dm), lambda i: (0, 0)),
            pl.BlockSpec((dm, pad), lambda i: (0, 0)),
            pl.BlockSpec((1, pad), lambda i: (0, 0)),
        ],
        out_specs=pl.BlockSpec((_BN, pad), lambda i: (i, 0)),
        out_shape=jax.ShapeDtypeStruct((n, pad), jnp.float32),
    )(h, stats, g.reshape(1, k), be.reshape(1, k), wl1, bl1.reshape(1, dm),
      wl2p, bl2p)
    return out[:, :d_out]


def kernel(x, edge_index, edge_weight, W1, b1, g1, be1, W2, b2, g2, be2,
           Wl1, bl1, Wl2, bl2):
    n, d_in = x.shape
    e = edge_weight.shape[0]
    row, col = edge_index[0], edge_index[1]

    degp = _deg_partials(col, edge_weight, n, e)
    dis, xs = _prep(degp, x)

    agg1p = _sc_aggregate(xs, row, col, edge_weight, n, e, 1)
    h1, st1 = _mm_stats(agg1p[:, 0], xs, dis, W1, b1)

    hs2 = _bn_mm(h1, st1, g1, be1, W2, dis)
    agg2p = _sc_aggregate(hs2.reshape(n * 8, 128), row, col, edge_weight,
                          n, e, 8)
    h2, st2 = _comb_stats(agg2p.reshape(16, n, 128), hs2, dis, b2)
    return _head(h2, st2, g2, be2, Wl1, bl1, Wl2, bl2, Wl2.shape[1])


# speedup vs baseline: 9.1582x; 1.6578x over previous
import functools

import jax
import jax.numpy as jnp
from jax import lax
from jax.experimental import pallas as pl
from jax.experimental.pallas import tpu as pltpu
from jax.experimental.pallas import tpu_sc as plsc

_EPS = 1e-5
_BN = 1000
_NW = 32
_NSUB = 16
_B = 80
_ACC_PAD = 10240


def _deg_body(col_hbm, ew_hbm, out_hbm, cbuf, ebuf, dbuf, *, n, e):
    epw = e // _NW
    w = lax.axis_index("s") * 2 + lax.axis_index("c")
    base = pl.multiple_of(w * epw, 8)
    pltpu.sync_copy(col_hbm.at[pl.ds(base, epw)], cbuf)
    pltpu.sync_copy(ew_hbm.at[pl.ds(base, epw)], ebuf)

    zv = jnp.zeros((16,), jnp.float32)

    def zero(k, _):
        dbuf[pl.ds(k * 16, 16)] = zv
        return ()
    lax.fori_loop(0, n // 16, zero, ())

    def body(k, _):
        cv = cbuf[pl.ds(k * 16, 16)]
        ev = ebuf[pl.ds(k * 16, 16)]
        plsc.addupdate_scatter(dbuf, [cv], ev)
        return ()
    lax.fori_loop(0, epw // 16, body, ())
    pltpu.sync_copy(dbuf, out_hbm.at[w])


def _deg_partials(col, ew, n, e):
    epw = e // _NW
    mesh = plsc.VectorSubcoreMesh(core_axis_name="c", subcore_axis_name="s")
    return pl.kernel(
        functools.partial(_deg_body, n=n, e=e),
        out_type=jax.ShapeDtypeStruct((_NW, n), jnp.float32),
        mesh=mesh,
        compiler_params=pltpu.CompilerParams(needs_layout_passes=False),
        scratch_types=[
            pltpu.VMEM((epw,), jnp.int32),
            pltpu.VMEM((epw,), jnp.float32),
            pltpu.VMEM((n,), jnp.float32),
        ],
    )(col, ew)


def _agg_body(tab_hbm, row_hbm, col_hbm, ew_hbm, out_hbm,
              rbuf, cbuf, cblk0, cblk1, ewb0, ewb1, gbuf0, gbuf1,
              acc, gsem0, gsem1, ssem0, ssem1,
              *, n, e, nch):
    epw = e // _NW
    nblk = epw // _B
    core = lax.axis_index("c")
    s = lax.axis_index("s")
    w = s * 2 + core
    ebase = pl.multiple_of(w * epw, 8)

    pltpu.sync_copy(row_hbm.at[pl.ds(ebase, epw)], rbuf)
    pltpu.sync_copy(col_hbm.at[pl.ds(ebase, epw)], cbuf)

    cblk = (cblk0, cblk1)
    ewb = (ewb0, ewb1)
    gbuf = (gbuf0, gbuf1)
    gsem = (gsem0, gsem1)
    ssem = (ssem0, ssem1)

    zv = jnp.zeros((16,), jnp.float32)
    my0 = 640 * s
    last = n - 640 * (_NSUB - 1)

    if nch > 1:
        def adj(b, _):
            o = b * 16
            rbuf[pl.ds(o, 16)] = rbuf[pl.ds(o, 16)] * nch
            return ()
        lax.fori_loop(0, epw // 16, adj, ())

    def issue_gather(b, kk):
        off = pl.multiple_of(b * _B, 8)
        pltpu.async_copy(tab_hbm.at[rbuf.at[pl.ds(off, _B)]], gbuf[kk],
                         gsem[kk])
        pltpu.async_copy(ew_hbm.at[pl.ds(ebase + off, _B)], ewb[kk],
                         gsem[kk])

    def wait_gather(b, kk):
        off = pl.multiple_of(b * _B, 8)
        pltpu.make_async_copy(tab_hbm.at[rbuf.at[pl.ds(off, _B)]], gbuf[kk],
                              gsem[kk]).wait()
        pltpu.make_async_copy(ew_hbm.at[pl.ds(ebase + off, _B)], ewb[kk],
                              gsem[kk]).wait()

    def rebuild_cblk(b, kk):
        off = b * _B
        for t in range(_B // 16):
            cblk[kk][pl.ds(t * 16, 16)] = cbuf[pl.ds(off + t * 16, 16)]

    def issue_scatter(kk):
        pltpu.async_copy(gbuf[kk], acc.at[cblk[kk]], ssem[kk], add=True)

    def wait_scatter(kk):
        pltpu.make_async_copy(gbuf[kk], acc.at[cblk[kk]], ssem[kk]).wait()

    def scale(kk):
        def srow(i, _):
            sp = plsc.load_gather(ewb[kk], [jnp.full((16,), i, jnp.int32)])
            for j in range(8):
                gbuf[kk][i, pl.ds(j * 16, 16)] = (
                    gbuf[kk][i, pl.ds(j * 16, 16)] * sp)
            return ()
        lax.fori_loop(0, _B, srow, ())

    for ch in range(nch):
        if ch > 0:
            def bump(b, _):
                o = b * 16
                rbuf[pl.ds(o, 16)] = rbuf[pl.ds(o, 16)] + 1
                return ()
            lax.fori_loop(0, epw // 16, bump, ())

        def zrow(i, _):
            for j in range(8):
                gbuf0[i, pl.ds(j * 16, 16)] = zv
            return ()
        lax.fori_loop(0, _B, zrow, ())

        @pl.when(s != _NSUB - 1)
        def _():
            for j in range(8):
                pltpu.sync_copy(gbuf0, acc.at[pl.ds(my0 + j * _B, _B)])

        @pl.when(s == _NSUB - 1)
        def _():
            for j in range(last // _B):
                pltpu.sync_copy(gbuf0, acc.at[pl.ds(my0 + j * _B, _B)])
        plsc.subcore_barrier()

        rebuild_cblk(0, 0)
        issue_gather(0, 0)

        def do_block(b, cur, nxt):
            @pl.when(b + 1 < nblk)
            def _():
                @pl.when(b >= 1)
                def _():
                    wait_scatter(nxt)
                rebuild_cblk(b + 1, nxt)
                issue_gather(b + 1, nxt)
            wait_gather(b, cur)
            scale(cur)
            issue_scatter(cur)

        def body(b, _):
            @pl.when(b % 2 == 0)
            def _():
                do_block(b, 0, 1)

            @pl.when(b % 2 == 1)
            def _():
                do_block(b, 1, 0)
            return ()
        lax.fori_loop(0, nblk, body, ())
        wait_scatter((nblk - 2) % 2)
        wait_scatter((nblk - 1) % 2)
        plsc.subcore_barrier()

        @pl.when(s != _NSUB - 1)
        def _():
            pltpu.sync_copy(acc.at[pl.ds(my0, 640)],
                            out_hbm.at[core, ch, pl.ds(my0, 640)])

        @pl.when(s == _NSUB - 1)
        def _():
            pltpu.sync_copy(acc.at[pl.ds(my0, last)],
                            out_hbm.at[core, ch, pl.ds(my0, last)])


def _sc_aggregate(table, row, col, ew, n, e, nch):
    epw = e // _NW
    nblk = epw // _B
    mesh = plsc.VectorSubcoreMesh(core_axis_name="c", subcore_axis_name="s")
    return pl.kernel(
        functools.partial(_agg_body, n=n, e=e, nch=nch),
        out_type=jax.ShapeDtypeStruct((2, nch, n, 128), jnp.float32),
        mesh=mesh,
        compiler_params=pltpu.CompilerParams(needs_layout_passes=False),
        scratch_types=[
            pltpu.VMEM((epw,), jnp.int32),
            pltpu.VMEM((epw,), jnp.int32),
            pltpu.VMEM((_B,), jnp.int32),
            pltpu.VMEM((_B,), jnp.int32),
            pltpu.VMEM((_B,), jnp.float32),
            pltpu.VMEM((_B,), jnp.float32),
            pltpu.VMEM((_B, 128), jnp.float32),
            pltpu.VMEM((_B, 128), jnp.float32),
            pltpu.VMEM_SHARED((n, 128), jnp.float32),
            pltpu.SemaphoreType.DMA,
            pltpu.SemaphoreType.DMA,
            pltpu.SemaphoreType.DMA,
            pltpu.SemaphoreType.DMA,
        ],
    )(table, row, col, ew)


def _prep_body(degp_ref, x_ref, dis_ref, xs_ref, *, n):
    d = jnp.sum(degp_ref[...], axis=0, keepdims=True) + 1.0
    t = jnp.transpose(jax.lax.rsqrt(d), (1, 0))
    dis_ref[...] = t
    xs_ref[...] = x_ref[...] * t


def _prep(degp, x):
    n, k = x.shape
    return pl.pallas_call(
        functools.partial(_prep_body, n=n),
        grid=(1,),
        in_specs=[
            pl.BlockSpec((_NW, n), lambda i: (0, 0)),
            pl.BlockSpec((n, k), lambda i: (0, 0)),
        ],
        out_specs=[
            pl.BlockSpec((n, 1), lambda i: (0, 0)),
            pl.BlockSpec((n, k), lambda i: (0, 0)),
        ],
        out_shape=[
            jax.ShapeDtypeStruct((n, 1), jnp.float32),
            jax.ShapeDtypeStruct((n, k), jnp.float32),
        ],
    )(degp, x)


def _mm_stats_body(aggp_ref, xs_ref, dis_ref, w_ref, b_ref, h_ref, st_ref):
    i = pl.program_id(0)
    a = (aggp_ref[0] + aggp_ref[1] + xs_ref[...]) * dis_ref[...]
    h = jnp.dot(a, w_ref[...], preferred_element_type=jnp.float32)
    h = h + b_ref[...]
    h_ref[...] = h

    @pl.when(i == 0)
    def _():
        st_ref[...] = jnp.zeros_like(st_ref)

    st_ref[0, :] += jnp.sum(h, axis=0)
    st_ref[1, :] += jnp.sum(h * h, axis=0)


def _mm_stats(aggp, xs, dis, w, b):
    n, k = xs.shape
    m = w.shape[1]
    return pl.pallas_call(
        _mm_stats_body,
        grid=(n // _BN,),
        in_specs=[
            pl.BlockSpec((2, _BN, k), lambda i: (0, i, 0)),
            pl.BlockSpec((_BN, k), lambda i: (i, 0)),
            pl.BlockSpec((_BN, 1), lambda i: (i, 0)),
            pl.BlockSpec((k, m), lambda i: (0, 0)),
            pl.BlockSpec((1, m), lambda i: (0, 0)),
        ],
        out_specs=[
            pl.BlockSpec((_BN, m), lambda i: (i, 0)),
            pl.BlockSpec((2, m), lambda i: (0, 0)),
        ],
        out_shape=[
            jax.ShapeDtypeStruct((n, m), jnp.float32),
            jax.ShapeDtypeStruct((2, m), jnp.float32),
        ],
    )(aggp, xs, dis, w.reshape(k, m), b.reshape(1, m))


def _bn_mm_body(h_ref, st_ref, g_ref, be_ref, w_ref, dis_ref, hs_ref, *, n):
    mu = st_ref[0, :] * (1.0 / n)
    var = st_ref[1, :] * (1.0 / n) - mu * mu
    scale = jax.lax.rsqrt(var + _EPS) * g_ref[0, :]
    hn = jnp.maximum((h_ref[...] - mu[None, :]) * scale[None, :]
                     + be_ref[0, :][None, :], 0.0)
    hw = jnp.dot(hn, w_ref[...], preferred_element_type=jnp.float32)
    hs_ref[...] = hw * dis_ref[...]


def _bn_mm(h, stats, g, be, w, dis):
    n, k = h.shape
    m = w.shape[1]
    return pl.pallas_call(
        functools.partial(_bn_mm_body, n=n),
        grid=(n // _BN,),
        in_specs=[
            pl.BlockSpec((_BN, k), lambda i: (i, 0)),
            pl.BlockSpec((2, k), lambda i: (0, 0)),
            pl.BlockSpec((1, k), lambda i: (0, 0)),
            pl.BlockSpec((1, k), lambda i: (0, 0)),
            pl.BlockSpec((k, m), lambda i: (0, 0)),
            pl.BlockSpec((_BN, 1), lambda i: (i, 0)),
        ],
        out_specs=pl.BlockSpec((_BN, m), lambda i: (i, 0)),
        out_shape=jax.ShapeDtypeStruct((n, m), jnp.float32),
    )(h, stats, g.reshape(1, k), be.reshape(1, k), w, dis)


def _comb_stats_body(aggp_ref, hs_ref, dis_ref, b_ref, h_ref, st_ref):
    i = pl.program_id(0)
    agg = jnp.concatenate(
        [aggp_ref[c] + aggp_ref[8 + c] for c in range(8)], axis=1)
    h = (agg + hs_ref[...]) * dis_ref[...] + b_ref[...]
    h_ref[...] = h

    @pl.when(i == 0)
    def _():
        st_ref[...] = jnp.zeros_like(st_ref)

    st_ref[0, :] += jnp.sum(h, axis=0)
    st_ref[1, :] += jnp.sum(h * h, axis=0)


def _comb_stats(aggp, hs, dis, b):
    n, m = hs.shape
    return pl.pallas_call(
        _comb_stats_body,
        grid=(n // _BN,),
        in_specs=[
            pl.BlockSpec((16, _BN, 128), lambda i: (0, i, 0)),
            pl.BlockSpec((_BN, m), lambda i: (i, 0)),
            pl.BlockSpec((_BN, 1), lambda i: (i, 0)),
            pl.BlockSpec((1, m), lambda i: (0, 0)),
        ],
        out_specs=[
            pl.BlockSpec((_BN, m), lambda i: (i, 0)),
            pl.BlockSpec((2, m), lambda i: (0, 0)),
        ],
        out_shape=[
            jax.ShapeDtypeStruct((n, m), jnp.float32),
            jax.ShapeDtypeStruct((2, m), jnp.float32),
        ],
    )(aggp, hs, dis, b.reshape(1, m))


def _head_body(h_ref, st_ref, g_ref, be_ref, w1_ref, b1_ref, w2_ref, b2_ref,
               o_ref, *, n, d_out):
    mu = st_ref[0, :] * (1.0 / n)
    var = st_ref[1, :] * (1.0 / n) - mu * mu
    scale = jax.lax.rsqrt(var + _EPS) * g_ref[0, :]
    hn = jnp.maximum((h_ref[...] - mu[None, :]) * scale[None, :]
                     + be_ref[0, :][None, :], 0.0)
    m1 = jnp.maximum(jnp.dot(hn, w1_ref[...],
                             preferred_element_type=jnp.float32)
                     + b1_ref[...], 0.0)
    o = jnp.dot(m1, w2_ref[...], preferred_element_type=jnp.float32) + b2_ref[...]
    lane = jax.lax.broadcasted_iota(jnp.int32, o.shape, 1)
    valid = lane < d_out
    om = jnp.where(valid, o, -jnp.inf)
    mx = jnp.max(om, axis=1, keepdims=True)
    ex = jnp.where(valid, jnp.exp(om - mx), 0.0)
    lse = jnp.log(jnp.sum(ex, axis=1, keepdims=True)) + mx
    o_ref[...] = o - lse


def _head(h, stats, g, be, wl1, bl1, wl2, bl2, d_out):
    n, k = h.shape
    dm = wl1.shape[1]
    pad = 128
    wl2p = jnp.zeros((dm, pad), jnp.float32).at[:, :d_out].set(wl2)
    bl2p = jnp.zeros((1, pad), jnp.float32).at[0, :d_out].set(bl2)
    out = pl.pallas_call(
        functools.partial(_head_body, n=n, d_out=d_out),
        grid=(n // _BN,),
        in_specs=[
            pl.BlockSpec((_BN, k), lambda i: (i, 0)),
            pl.BlockSpec((2, k), lambda i: (0, 0)),
            pl.BlockSpec((1, k), lambda i: (0, 0)),
            pl.BlockSpec((1, k), lambda i: (0, 0)),
            pl.BlockSpec((k, dm), lambda i: (0, 0)),
            pl.BlockSpec((1, dm), lambda i: (0, 0)),
            pl.BlockSpec((dm, pad), lambda i: (0, 0)),
            pl.BlockSpec((1, pad), lambda i: (0, 0)),
        ],
        out_specs=pl.BlockSpec((_BN, pad), lambda i: (i, 0)),
        out_shape=jax.ShapeDtypeStruct((n, pad), jnp.float32),
    )(h, stats, g.reshape(1, k), be.reshape(1, k), wl1, bl1.reshape(1, dm),
      wl2p, bl2p)
    return out[:, :d_out]


def kernel(x, edge_index, edge_weight, W1, b1, g1, be1, W2, b2, g2, be2,
           Wl1, bl1, Wl2, bl2):
    n, d_in = x.shape
    e = edge_weight.shape[0]
    row, col = edge_index[0], edge_index[1]

    degp = _deg_partials(col, edge_weight, n, e)
    dis, xs = _prep(degp, x)

    agg1p = _sc_aggregate(xs, row, col, edge_weight, n, e, 1)
    h1, st1 = _mm_stats(agg1p[:, 0], xs, dis, W1, b1)

    hs2 = _bn_mm(h1, st1, g1, be1, W2, dis)
    agg2p = _sc_aggregate(hs2.reshape(n * 8, 128), row, col, edge_weight,
                          n, e, 8)
    h2, st2 = _comb_stats(agg2p.reshape(16, n, 128), hs2, dis, b2)
    return _head(h2, st2, g2, be2, Wl1, bl1, Wl2, bl2, Wl2.shape[1])


# speedup vs baseline: 10.9735x; 1.1982x over previous
import functools

import jax
import jax.numpy as jnp
from jax import lax
from jax.experimental import pallas as pl
from jax.experimental.pallas import tpu as pltpu
from jax.experimental.pallas import tpu_sc as plsc

_EPS = 1e-5
_BN = 1000
_NW = 32
_NSUB = 16
_B = 80
_ACC_PAD = 10240


def _deg_body(col_hbm, ew_hbm, out_hbm, cbuf, ebuf, dbuf, *, n, e):
    epw = e // _NW
    w = lax.axis_index("s") * 2 + lax.axis_index("c")
    base = pl.multiple_of(w * epw, 8)
    pltpu.sync_copy(col_hbm.at[pl.ds(base, epw)], cbuf)
    pltpu.sync_copy(ew_hbm.at[pl.ds(base, epw)], ebuf)

    zv = jnp.zeros((16,), jnp.float32)

    def zero(k, _):
        dbuf[pl.ds(k * 16, 16)] = zv
        return ()
    lax.fori_loop(0, n // 16, zero, ())

    def body(k, _):
        cv = cbuf[pl.ds(k * 16, 16)]
        ev = ebuf[pl.ds(k * 16, 16)]
        plsc.addupdate_scatter(dbuf, [cv], ev)
        return ()
    lax.fori_loop(0, epw // 16, body, ())
    pltpu.sync_copy(dbuf, out_hbm.at[w])


def _deg_partials(col, ew, n, e):
    epw = e // _NW
    mesh = plsc.VectorSubcoreMesh(core_axis_name="c", subcore_axis_name="s")
    return pl.kernel(
        functools.partial(_deg_body, n=n, e=e),
        out_type=jax.ShapeDtypeStruct((_NW, n), jnp.float32),
        mesh=mesh,
        compiler_params=pltpu.CompilerParams(needs_layout_passes=False),
        scratch_types=[
            pltpu.VMEM((epw,), jnp.int32),
            pltpu.VMEM((epw,), jnp.float32),
            pltpu.VMEM((n,), jnp.float32),
        ],
    )(col, ew)


def _agg_body(tab_hbm, row_hbm, col_hbm, ew_hbm, out_hbm,
              rbuf, cblk0, cblk1, cblk2, ewb0, ewb1, ewb2,
              gbuf0, gbuf1, gbuf2,
              acc, gsem0, gsem1, gsem2, ssem0, ssem1, ssem2,
              *, n, e, nch):
    epw = e // _NW
    nblk = epw // _B
    core = lax.axis_index("c")
    s = lax.axis_index("s")
    w = s * 2 + core
    ebase = pl.multiple_of(w * epw, 8)

    pltpu.sync_copy(row_hbm.at[pl.ds(ebase, epw)], rbuf)

    cblk = (cblk0, cblk1, cblk2)
    ewb = (ewb0, ewb1, ewb2)
    gbuf = (gbuf0, gbuf1, gbuf2)
    gsem = (gsem0, gsem1, gsem2)
    ssem = (ssem0, ssem1, ssem2)

    zv = jnp.zeros((16,), jnp.float32)
    my0 = 640 * s
    last = n - 640 * (_NSUB - 1)

    if nch > 1:
        def adj(b, _):
            o = b * 16
            rbuf[pl.ds(o, 16)] = rbuf[pl.ds(o, 16)] * nch
            return ()
        lax.fori_loop(0, epw // 16, adj, ())

    def issue_gather(b, kk):
        off = pl.multiple_of(b * _B, 8)
        pltpu.async_copy(tab_hbm.at[rbuf.at[pl.ds(off, _B)]], gbuf[kk],
                         gsem[kk])
        pltpu.async_copy(ew_hbm.at[pl.ds(ebase + off, _B)], ewb[kk],
                         gsem[kk])
        pltpu.async_copy(col_hbm.at[pl.ds(ebase + off, _B)], cblk[kk],
                         gsem[kk])

    def wait_gather(b, kk):
        off = pl.multiple_of(b * _B, 8)
        pltpu.make_async_copy(tab_hbm.at[rbuf.at[pl.ds(off, _B)]], gbuf[kk],
                              gsem[kk]).wait()
        pltpu.make_async_copy(ew_hbm.at[pl.ds(ebase + off, _B)], ewb[kk],
                              gsem[kk]).wait()
        pltpu.make_async_copy(col_hbm.at[pl.ds(ebase + off, _B)], cblk[kk],
                              gsem[kk]).wait()

    def issue_scatter(kk):
        pltpu.async_copy(gbuf[kk], acc.at[cblk[kk]], ssem[kk], add=True)

    def wait_scatter(kk):
        pltpu.make_async_copy(gbuf[kk], acc.at[cblk[kk]], ssem[kk]).wait()

    def scale(kk):
        def srow(i2, _):
            i = i2 * 2
            sp0 = plsc.load_gather(ewb[kk], [jnp.full((16,), i, jnp.int32)])
            sp1 = plsc.load_gather(ewb[kk], [jnp.full((16,), i + 1,
                                                      jnp.int32)])
            for j in range(8):
                gbuf[kk][i, pl.ds(j * 16, 16)] = (
                    gbuf[kk][i, pl.ds(j * 16, 16)] * sp0)
            for j in range(8):
                gbuf[kk][i + 1, pl.ds(j * 16, 16)] = (
                    gbuf[kk][i + 1, pl.ds(j * 16, 16)] * sp1)
            return ()
        lax.fori_loop(0, _B // 2, srow, ())

    for ch in range(nch):
        if ch > 0:
            def bump(b, _):
                o = b * 16
                rbuf[pl.ds(o, 16)] = rbuf[pl.ds(o, 16)] + 1
                return ()
            lax.fori_loop(0, epw // 16, bump, ())

        def zrow(i, _):
            for j in range(8):
                gbuf0[i, pl.ds(j * 16, 16)] = zv
            return ()
        lax.fori_loop(0, _B, zrow, ())

        @pl.when(s != _NSUB - 1)
        def _():
            for j in range(8):
                pltpu.sync_copy(gbuf0, acc.at[pl.ds(my0 + j * _B, _B)])

        @pl.when(s == _NSUB - 1)
        def _():
            for j in range(last // _B):
                pltpu.sync_copy(gbuf0, acc.at[pl.ds(my0 + j * _B, _B)])
        plsc.subcore_barrier()

        issue_gather(0, 0)
        issue_gather(1, 1)

        def do_block(b, cur, nxt):
            @pl.when(b + 2 < nblk)
            def _():
                @pl.when(b >= 1)
                def _():
                    wait_scatter(nxt)
                issue_gather(b + 2, nxt)
            wait_gather(b, cur)
            scale(cur)
            issue_scatter(cur)

        def body(b, _):
            for r in range(3):
                @pl.when(b % 3 == r)
                def _(r=r):
                    do_block(b, r, (r + 2) % 3)
            return ()
        lax.fori_loop(0, nblk, body, ())
        wait_scatter((nblk - 3) % 3)
        wait_scatter((nblk - 2) % 3)
        wait_scatter((nblk - 1) % 3)
        plsc.subcore_barrier()

        @pl.when(s != _NSUB - 1)
        def _():
            pltpu.sync_copy(acc.at[pl.ds(my0, 640)],
                            out_hbm.at[core, ch, pl.ds(my0, 640)])

        @pl.when(s == _NSUB - 1)
        def _():
            pltpu.sync_copy(acc.at[pl.ds(my0, last)],
                            out_hbm.at[core, ch, pl.ds(my0, last)])


def _sc_aggregate(table, row, col, ew, n, e, nch):
    epw = e // _NW
    nblk = epw // _B
    mesh = plsc.VectorSubcoreMesh(core_axis_name="c", subcore_axis_name="s")
    return pl.kernel(
        functools.partial(_agg_body, n=n, e=e, nch=nch),
        out_type=jax.ShapeDtypeStruct((2, nch, n, 128), jnp.float32),
        mesh=mesh,
        compiler_params=pltpu.CompilerParams(needs_layout_passes=False),
        scratch_types=[
            pltpu.VMEM((epw,), jnp.int32),
            pltpu.VMEM((_B,), jnp.int32),
            pltpu.VMEM((_B,), jnp.int32),
            pltpu.VMEM((_B,), jnp.int32),
            pltpu.VMEM((_B,), jnp.float32),
            pltpu.VMEM((_B,), jnp.float32),
            pltpu.VMEM((_B,), jnp.float32),
            pltpu.VMEM((_B, 128), jnp.float32),
            pltpu.VMEM((_B, 128), jnp.float32),
            pltpu.VMEM((_B, 128), jnp.float32),
            pltpu.VMEM_SHARED((n, 128), jnp.float32),
            pltpu.SemaphoreType.DMA,
            pltpu.SemaphoreType.DMA,
            pltpu.SemaphoreType.DMA,
            pltpu.SemaphoreType.DMA,
            pltpu.SemaphoreType.DMA,
            pltpu.SemaphoreType.DMA,
        ],
    )(table, row, col, ew)


def _prep_body(degp_ref, x_ref, dis_ref, xs_ref, *, n):
    d = jnp.sum(degp_ref[...], axis=0, keepdims=True) + 1.0
    t = jnp.transpose(jax.lax.rsqrt(d), (1, 0))
    dis_ref[...] = t
    xs_ref[...] = x_ref[...] * t


def _prep(degp, x):
    n, k = x.shape
    return pl.pallas_call(
        functools.partial(_prep_body, n=n),
        grid=(1,),
        in_specs=[
            pl.BlockSpec((_NW, n), lambda i: (0, 0)),
            pl.BlockSpec((n, k), lambda i: (0, 0)),
        ],
        out_specs=[
            pl.BlockSpec((n, 1), lambda i: (0, 0)),
            pl.BlockSpec((n, k), lambda i: (0, 0)),
        ],
        out_shape=[
            jax.ShapeDtypeStruct((n, 1), jnp.float32),
            jax.ShapeDtypeStruct((n, k), jnp.float32),
        ],
    )(degp, x)


def _mm_stats_body(aggp_ref, xs_ref, dis_ref, w_ref, b_ref, h_ref, st_ref):
    i = pl.program_id(0)
    a = (aggp_ref[0] + aggp_ref[1] + xs_ref[...]) * dis_ref[...]
    h = jnp.dot(a, w_ref[...], preferred_element_type=jnp.float32)
    h = h + b_ref[...]
    h_ref[...] = h

    @pl.when(i == 0)
    def _():
        st_ref[...] = jnp.zeros_like(st_ref)

    st_ref[0, :] += jnp.sum(h, axis=0)
    st_ref[1, :] += jnp.sum(h * h, axis=0)


def _mm_stats(aggp, xs, dis, w, b):
    n, k = xs.shape
    m = w.shape[1]
    return pl.pallas_call(
        _mm_stats_body,
        grid=(n // _BN,),
        in_specs=[
            pl.BlockSpec((2, _BN, k), lambda i: (0, i, 0)),
            pl.BlockSpec((_BN, k), lambda i: (i, 0)),
            pl.BlockSpec((_BN, 1), lambda i: (i, 0)),
            pl.BlockSpec((k, m), lambda i: (0, 0)),
            pl.BlockSpec((1, m), lambda i: (0, 0)),
        ],
        out_specs=[
            pl.BlockSpec((_BN, m), lambda i: (i, 0)),
            pl.BlockSpec((2, m), lambda i: (0, 0)),
        ],
        out_shape=[
            jax.ShapeDtypeStruct((n, m), jnp.float32),
            jax.ShapeDtypeStruct((2, m), jnp.float32),
        ],
    )(aggp, xs, dis, w.reshape(k, m), b.reshape(1, m))


def _bn_mm_body(h_ref, st_ref, g_ref, be_ref, w_ref, dis_ref, hs_ref, *, n):
    mu = st_ref[0, :] * (1.0 / n)
    var = st_ref[1, :] * (1.0 / n) - mu * mu
    scale = jax.lax.rsqrt(var + _EPS) * g_ref[0, :]
    hn = jnp.maximum((h_ref[...] - mu[None, :]) * scale[None, :]
                     + be_ref[0, :][None, :], 0.0)
    hw = jnp.dot(hn, w_ref[...], preferred_element_type=jnp.float32)
    hs_ref[...] = hw * dis_ref[...]


def _bn_mm(h, stats, g, be, w, dis):
    n, k = h.shape
    m = w.shape[1]
    return pl.pallas_call(
        functools.partial(_bn_mm_body, n=n),
        grid=(n // _BN,),
        in_specs=[
            pl.BlockSpec((_BN, k), lambda i: (i, 0)),
            pl.BlockSpec((2, k), lambda i: (0, 0)),
            pl.BlockSpec((1, k), lambda i: (0, 0)),
            pl.BlockSpec((1, k), lambda i: (0, 0)),
            pl.BlockSpec((k, m), lambda i: (0, 0)),
            pl.BlockSpec((_BN, 1), lambda i: (i, 0)),
        ],
        out_specs=pl.BlockSpec((_BN, m), lambda i: (i, 0)),
        out_shape=jax.ShapeDtypeStruct((n, m), jnp.float32),
    )(h, stats, g.reshape(1, k), be.reshape(1, k), w, dis)


def _comb_stats_body(aggp_ref, hs_ref, dis_ref, b_ref, h_ref, st_ref):
    i = pl.program_id(0)
    agg = jnp.concatenate(
        [aggp_ref[c] + aggp_ref[8 + c] for c in range(8)], axis=1)
    h = (agg + hs_ref[...]) * dis_ref[...] + b_ref[...]
    h_ref[...] = h

    @pl.when(i == 0)
    def _():
        st_ref[...] = jnp.zeros_like(st_ref)

    st_ref[0, :] += jnp.sum(h, axis=0)
    st_ref[1, :] += jnp.sum(h * h, axis=0)


def _comb_stats(aggp, hs, dis, b):
    n, m = hs.shape
    return pl.pallas_call(
        _comb_stats_body,
        grid=(n // _BN,),
        in_specs=[
            pl.BlockSpec((16, _BN, 128), lambda i: (0, i, 0)),
            pl.BlockSpec((_BN, m), lambda i: (i, 0)),
            pl.BlockSpec((_BN, 1), lambda i: (i, 0)),
            pl.BlockSpec((1, m), lambda i: (0, 0)),
        ],
        out_specs=[
            pl.BlockSpec((_BN, m), lambda i: (i, 0)),
            pl.BlockSpec((2, m), lambda i: (0, 0)),
        ],
        out_shape=[
            jax.ShapeDtypeStruct((n, m), jnp.float32),
            jax.ShapeDtypeStruct((2, m), jnp.float32),
        ],
    )(aggp, hs, dis, b.reshape(1, m))


def _head_body(h_ref, st_ref, g_ref, be_ref, w1_ref, b1_ref, w2_ref, b2_ref,
               o_ref, *, n, d_out):
    mu = st_ref[0, :] * (1.0 / n)
    var = st_ref[1, :] * (1.0 / n) - mu * mu
    scale = jax.lax.rsqrt(var + _EPS) * g_ref[0, :]
    hn = jnp.maximum((h_ref[...] - mu[None, :]) * scale[None, :]
                     + be_ref[0, :][None, :], 0.0)
    m1 = jnp.maximum(jnp.dot(hn, w1_ref[...],
                             preferred_element_type=jnp.float32)
                     + b1_ref[...], 0.0)
    o = jnp.dot(m1, w2_ref[...], preferred_element_type=jnp.float32) + b2_ref[...]
    lane = jax.lax.broadcasted_iota(jnp.int32, o.shape, 1)
    valid = lane < d_out
    om = jnp.where(valid, o, -jnp.inf)
    mx = jnp.max(om, axis=1, keepdims=True)
    ex = jnp.where(valid, jnp.exp(om - mx), 0.0)
    lse = jnp.log(jnp.sum(ex, axis=1, keepdims=True)) + mx
    o_ref[...] = o - lse


def _head(h, stats, g, be, wl1, bl1, wl2, bl2, d_out):
    n, k = h.shape
    dm = wl1.shape[1]
    pad = 128
    wl2p = jnp.zeros((dm, pad), jnp.float32).at[:, :d_out].set(wl2)
    bl2p = jnp.zeros((1, pad), jnp.float32).at[0, :d_out].set(bl2)
    out = pl.pallas_call(
        functools.partial(_head_body, n=n, d_out=d_out),
        grid=(n // _BN,),
        in_specs=[
            pl.BlockSpec((_BN, k), lambda i: (i, 0)),
            pl.BlockSpec((2, k), lambda i: (0, 0)),
            pl.BlockSpec((1, k), lambda i: (0, 0)),
            pl.BlockSpec((1, k), lambda i: (0, 0)),
            pl.BlockSpec((k, dm), lambda i: (0, 0)),
            pl.BlockSpec((1, dm), lambda i: (0, 0)),
            pl.BlockSpec((dm, pad), lambda i: (0, 0)),
            pl.BlockSpec((1, pad), lambda i: (0, 0)),
        ],
        out_specs=pl.BlockSpec((_BN, pad), lambda i: (i, 0)),
        out_shape=jax.ShapeDtypeStruct((n, pad), jnp.float32),
    )(h, stats, g.reshape(1, k), be.reshape(1, k), wl1, bl1.reshape(1, dm),
      wl2p, bl2p)
    return out[:, :d_out]


def kernel(x, edge_index, edge_weight, W1, b1, g1, be1, W2, b2, g2, be2,
           Wl1, bl1, Wl2, bl2):
    n, d_in = x.shape
    e = edge_weight.shape[0]
    row, col = edge_index[0], edge_index[1]

    degp = _deg_partials(col, edge_weight, n, e)
    dis, xs = _prep(degp, x)

    agg1p = _sc_aggregate(xs, row, col, edge_weight, n, e, 1)
    h1, st1 = _mm_stats(agg1p[:, 0], xs, dis, W1, b1)

    hs2 = _bn_mm(h1, st1, g1, be1, W2, dis)
    agg2p = _sc_aggregate(hs2.reshape(n * 8, 128), row, col, edge_weight,
                          n, e, 8)
    h2, st2 = _comb_stats(agg2p.reshape(16, n, 128), hs2, dis, b2)
    return _head(h2, st2, g2, be2, Wl1, bl1, Wl2, bl2, Wl2.shape[1])


# speedup vs baseline: 11.9198x; 1.0862x over previous
import functools

import jax
import jax.numpy as jnp
from jax import lax
from jax.experimental import pallas as pl
from jax.experimental.pallas import tpu as pltpu
from jax.experimental.pallas import tpu_sc as plsc

_EPS = 1e-5
_BN = 1000
_NW = 32
_NSUB = 16
_B = 80
_ACC_PAD = 10240


def _deg_body(col_hbm, ew_hbm, out_hbm, cbuf, ebuf, dbuf, *, n, e):
    epw = e // _NW
    w = lax.axis_index("s") * 2 + lax.axis_index("c")
    base = pl.multiple_of(w * epw, 8)
    pltpu.sync_copy(col_hbm.at[pl.ds(base, epw)], cbuf)
    pltpu.sync_copy(ew_hbm.at[pl.ds(base, epw)], ebuf)

    zv = jnp.zeros((16,), jnp.float32)

    def zero(k, _):
        dbuf[pl.ds(k * 16, 16)] = zv
        return ()
    lax.fori_loop(0, n // 16, zero, ())

    def body(k, _):
        cv = cbuf[pl.ds(k * 16, 16)]
        ev = ebuf[pl.ds(k * 16, 16)]
        plsc.addupdate_scatter(dbuf, [cv], ev)
        return ()
    lax.fori_loop(0, epw // 16, body, ())
    pltpu.sync_copy(dbuf, out_hbm.at[w])


def _deg_partials(col, ew, n, e):
    epw = e // _NW
    mesh = plsc.VectorSubcoreMesh(core_axis_name="c", subcore_axis_name="s")
    return pl.kernel(
        functools.partial(_deg_body, n=n, e=e),
        out_type=jax.ShapeDtypeStruct((_NW, n), jnp.float32),
        mesh=mesh,
        compiler_params=pltpu.CompilerParams(needs_layout_passes=False),
        scratch_types=[
            pltpu.VMEM((epw,), jnp.int32),
            pltpu.VMEM((epw,), jnp.float32),
            pltpu.VMEM((n,), jnp.float32),
        ],
    )(col, ew)


def _agg_body(tab_hbm, row_hbm, col_hbm, ew_hbm, out_hbm,
              rib0, rib1, rib2, rib3, cblk0, cblk1, cblk2, cblk3,
              ewb0, ewb1, ewb2, ewb3, gbuf0, gbuf1, gbuf2, gbuf3,
              acc, isem0, isem1, isem2, isem3,
              gsem0, gsem1, gsem2, gsem3, ssem0, ssem1, ssem2, ssem3,
              *, n, e, nch):
    epw = e // _NW
    nblk = epw // _B
    core = lax.axis_index("c")
    s = lax.axis_index("s")
    w = s * 2 + core
    ebase = pl.multiple_of(w * epw, 8)

    rib = (rib0, rib1, rib2, rib3)
    cblk = (cblk0, cblk1, cblk2, cblk3)
    ewb = (ewb0, ewb1, ewb2, ewb3)
    gbuf = (gbuf0, gbuf1, gbuf2, gbuf3)
    isem = (isem0, isem1, isem2, isem3)
    gsem = (gsem0, gsem1, gsem2, gsem3)
    ssem = (ssem0, ssem1, ssem2, ssem3)

    zv = jnp.zeros((16,), jnp.float32)
    my0 = 640 * s
    last = n - 640 * (_NSUB - 1)

    def issue_idx(b, kk):
        off = pl.multiple_of(b * _B, 8)
        pltpu.async_copy(row_hbm.at[pl.ds(ebase + off, _B)], rib[kk],
                         isem[kk])
        pltpu.async_copy(col_hbm.at[pl.ds(ebase + off, _B)], cblk[kk],
                         isem[kk])
        pltpu.async_copy(ew_hbm.at[pl.ds(ebase + off, _B)], ewb[kk],
                         isem[kk])

    def wait_idx(b, kk):
        off = pl.multiple_of(b * _B, 8)
        pltpu.make_async_copy(row_hbm.at[pl.ds(ebase + off, _B)], rib[kk],
                              isem[kk]).wait()
        pltpu.make_async_copy(col_hbm.at[pl.ds(ebase + off, _B)], cblk[kk],
                              isem[kk]).wait()
        pltpu.make_async_copy(ew_hbm.at[pl.ds(ebase + off, _B)], ewb[kk],
                              isem[kk]).wait()

    def adjust(kk, ch):
        if nch > 1:
            for t in range(_B // 16):
                rib[kk][pl.ds(t * 16, 16)] = (
                    rib[kk][pl.ds(t * 16, 16)] * nch + ch)

    def issue_gather(kk):
        pltpu.async_copy(tab_hbm.at[rib[kk]], gbuf[kk], gsem[kk])

    def wait_gather(kk):
        pltpu.make_async_copy(tab_hbm.at[rib[kk]], gbuf[kk],
                              gsem[kk]).wait()

    def issue_scatter(kk):
        pltpu.async_copy(gbuf[kk], acc.at[cblk[kk]], ssem[kk], add=True)

    def wait_scatter(kk):
        pltpu.make_async_copy(gbuf[kk], acc.at[cblk[kk]], ssem[kk]).wait()

    def scale(kk):
        def srow(i2, _):
            i = i2 * 2
            sp0 = plsc.load_gather(ewb[kk], [jnp.full((16,), i, jnp.int32)])
            sp1 = plsc.load_gather(ewb[kk], [jnp.full((16,), i + 1,
                                                      jnp.int32)])
            for j in range(8):
                gbuf[kk][i, pl.ds(j * 16, 16)] = (
                    gbuf[kk][i, pl.ds(j * 16, 16)] * sp0)
            for j in range(8):
                gbuf[kk][i + 1, pl.ds(j * 16, 16)] = (
                    gbuf[kk][i + 1, pl.ds(j * 16, 16)] * sp1)
            return ()
        lax.fori_loop(0, _B // 2, srow, ())

    for ch in range(nch):
        def zrow(i, _):
            for j in range(8):
                gbuf0[i, pl.ds(j * 16, 16)] = zv
            return ()
        lax.fori_loop(0, _B, zrow, ())

        @pl.when(s != _NSUB - 1)
        def _():
            for j in range(8):
                pltpu.sync_copy(gbuf0, acc.at[pl.ds(my0 + j * _B, _B)])

        @pl.when(s == _NSUB - 1)
        def _():
            for j in range(last // _B):
                pltpu.sync_copy(gbuf0, acc.at[pl.ds(my0 + j * _B, _B)])
        plsc.subcore_barrier()

        issue_idx(0, 0)
        issue_idx(1, 1)
        wait_idx(0, 0)
        adjust(0, ch)
        issue_gather(0)

        def do_block(b, cur):
            s1 = (cur + 1) % 4
            s2 = (cur + 2) % 4

            @pl.when(b + 2 < nblk)
            def _():
                @pl.when(b >= 2)
                def _():
                    wait_scatter(s2)
                issue_idx(b + 2, s2)

            @pl.when(b + 1 < nblk)
            def _():
                wait_idx(b + 1, s1)
                adjust(s1, ch)
                issue_gather(s1)
            wait_gather(cur)
            scale(cur)
            issue_scatter(cur)

        def body(b, _):
            for r in range(4):
                @pl.when(b % 4 == r)
                def _(r=r):
                    do_block(b, r)
            return ()
        lax.fori_loop(0, nblk, body, ())
        for q in range(4, 0, -1):
            wait_scatter((nblk - q) % 4)
        plsc.subcore_barrier()

        @pl.when(s != _NSUB - 1)
        def _():
            pltpu.sync_copy(acc.at[pl.ds(my0, 640)],
                            out_hbm.at[core, ch, pl.ds(my0, 640)])

        @pl.when(s == _NSUB - 1)
        def _():
            pltpu.sync_copy(acc.at[pl.ds(my0, last)],
                            out_hbm.at[core, ch, pl.ds(my0, last)])


def _sc_aggregate(table, row, col, ew, n, e, nch):
    epw = e // _NW
    nblk = epw // _B
    mesh = plsc.VectorSubcoreMesh(core_axis_name="c", subcore_axis_name="s")
    return pl.kernel(
        functools.partial(_agg_body, n=n, e=e, nch=nch),
        out_type=jax.ShapeDtypeStruct((2, nch, n, 128), jnp.float32),
        mesh=mesh,
        compiler_params=pltpu.CompilerParams(needs_layout_passes=False),
        scratch_types=(
            [pltpu.VMEM((_B,), jnp.int32) for _ in range(8)]
            + [pltpu.VMEM((_B,), jnp.float32) for _ in range(4)]
            + [pltpu.VMEM((_B, 128), jnp.float32) for _ in range(4)]
            + [pltpu.VMEM_SHARED((n, 128), jnp.float32)]
            + [pltpu.SemaphoreType.DMA for _ in range(12)]
        ),
    )(table, row, col, ew)


def _prep_body(degp_ref, x_ref, dis_ref, xs_ref, *, n):
    d = jnp.sum(degp_ref[...], axis=0, keepdims=True) + 1.0
    t = jnp.transpose(jax.lax.rsqrt(d), (1, 0))
    dis_ref[...] = t
    xs_ref[...] = x_ref[...] * t


def _prep(degp, x):
    n, k = x.shape
    return pl.pallas_call(
        functools.partial(_prep_body, n=n),
        grid=(1,),
        in_specs=[
            pl.BlockSpec((_NW, n), lambda i: (0, 0)),
            pl.BlockSpec((n, k), lambda i: (0, 0)),
        ],
        out_specs=[
            pl.BlockSpec((n, 1), lambda i: (0, 0)),
            pl.BlockSpec((n, k), lambda i: (0, 0)),
        ],
        out_shape=[
            jax.ShapeDtypeStruct((n, 1), jnp.float32),
            jax.ShapeDtypeStruct((n, k), jnp.float32),
        ],
    )(degp, x)


def _mm_stats_body(aggp_ref, xs_ref, dis_ref, w_ref, b_ref, h_ref, st_ref):
    i = pl.program_id(0)
    a = (aggp_ref[0] + aggp_ref[1] + xs_ref[...]) * dis_ref[...]
    h = jnp.dot(a, w_ref[...], preferred_element_type=jnp.float32)
    h = h + b_ref[...]
    h_ref[...] = h

    @pl.when(i == 0)
    def _():
        st_ref[...] = jnp.zeros_like(st_ref)

    st_ref[0, :] += jnp.sum(h, axis=0)
    st_ref[1, :] += jnp.sum(h * h, axis=0)


def _mm_stats(aggp, xs, dis, w, b):
    n, k = xs.shape
    m = w.shape[1]
    return pl.pallas_call(
        _mm_stats_body,
        grid=(n // _BN,),
        in_specs=[
            pl.BlockSpec((2, _BN, k), lambda i: (0, i, 0)),
            pl.BlockSpec((_BN, k), lambda i: (i, 0)),
            pl.BlockSpec((_BN, 1), lambda i: (i, 0)),
            pl.BlockSpec((k, m), lambda i: (0, 0)),
            pl.BlockSpec((1, m), lambda i: (0, 0)),
        ],
        out_specs=[
            pl.BlockSpec((_BN, m), lambda i: (i, 0)),
            pl.BlockSpec((2, m), lambda i: (0, 0)),
        ],
        out_shape=[
            jax.ShapeDtypeStruct((n, m), jnp.float32),
            jax.ShapeDtypeStruct((2, m), jnp.float32),
        ],
    )(aggp, xs, dis, w.reshape(k, m), b.reshape(1, m))


def _bn_mm_body(h_ref, st_ref, g_ref, be_ref, w_ref, dis_ref, hs_ref, *, n):
    mu = st_ref[0, :] * (1.0 / n)
    var = st_ref[1, :] * (1.0 / n) - mu * mu
    scale = jax.lax.rsqrt(var + _EPS) * g_ref[0, :]
    hn = jnp.maximum((h_ref[...] - mu[None, :]) * scale[None, :]
                     + be_ref[0, :][None, :], 0.0)
    hw = jnp.dot(hn, w_ref[...], preferred_element_type=jnp.float32)
    hs_ref[...] = hw * dis_ref[...]


def _bn_mm(h, stats, g, be, w, dis):
    n, k = h.shape
    m = w.shape[1]
    return pl.pallas_call(
        functools.partial(_bn_mm_body, n=n),
        grid=(n // _BN,),
        in_specs=[
            pl.BlockSpec((_BN, k), lambda i: (i, 0)),
            pl.BlockSpec((2, k), lambda i: (0, 0)),
            pl.BlockSpec((1, k), lambda i: (0, 0)),
            pl.BlockSpec((1, k), lambda i: (0, 0)),
            pl.BlockSpec((k, m), lambda i: (0, 0)),
            pl.BlockSpec((_BN, 1), lambda i: (i, 0)),
        ],
        out_specs=pl.BlockSpec((_BN, m), lambda i: (i, 0)),
        out_shape=jax.ShapeDtypeStruct((n, m), jnp.float32),
    )(h, stats, g.reshape(1, k), be.reshape(1, k), w, dis)


def _comb_stats_body(aggp_ref, hs_ref, dis_ref, b_ref, h_ref, st_ref):
    i = pl.program_id(0)
    agg = jnp.concatenate(
        [aggp_ref[c] + aggp_ref[8 + c] for c in range(8)], axis=1)
    h = (agg + hs_ref[...]) * dis_ref[...] + b_ref[...]
    h_ref[...] = h

    @pl.when(i == 0)
    def _():
        st_ref[...] = jnp.zeros_like(st_ref)

    st_ref[0, :] += jnp.sum(h, axis=0)
    st_ref[1, :] += jnp.sum(h * h, axis=0)


def _comb_stats(aggp, hs, dis, b):
    n, m = hs.shape
    return pl.pallas_call(
        _comb_stats_body,
        grid=(n // _BN,),
        in_specs=[
            pl.BlockSpec((16, _BN, 128), lambda i: (0, i, 0)),
            pl.BlockSpec((_BN, m), lambda i: (i, 0)),
            pl.BlockSpec((_BN, 1), lambda i: (i, 0)),
            pl.BlockSpec((1, m), lambda i: (0, 0)),
        ],
        out_specs=[
            pl.BlockSpec((_BN, m), lambda i: (i, 0)),
            pl.BlockSpec((2, m), lambda i: (0, 0)),
        ],
        out_shape=[
            jax.ShapeDtypeStruct((n, m), jnp.float32),
            jax.ShapeDtypeStruct((2, m), jnp.float32),
        ],
    )(aggp, hs, dis, b.reshape(1, m))


def _head_body(h_ref, st_ref, g_ref, be_ref, w1_ref, b1_ref, w2_ref, b2_ref,
               o_ref, *, n, d_out):
    mu = st_ref[0, :] * (1.0 / n)
    var = st_ref[1, :] * (1.0 / n) - mu * mu
    scale = jax.lax.rsqrt(var + _EPS) * g_ref[0, :]
    hn = jnp.maximum((h_ref[...] - mu[None, :]) * scale[None, :]
                     + be_ref[0, :][None, :], 0.0)
    m1 = jnp.maximum(jnp.dot(hn, w1_ref[...],
                             preferred_element_type=jnp.float32)
                     + b1_ref[...], 0.0)
    o = jnp.dot(m1, w2_ref[...], preferred_element_type=jnp.float32) + b2_ref[...]
    lane = jax.lax.broadcasted_iota(jnp.int32, o.shape, 1)
    valid = lane < d_out
    om = jnp.where(valid, o, -jnp.inf)
    mx = jnp.max(om, axis=1, keepdims=True)
    ex = jnp.where(valid, jnp.exp(om - mx), 0.0)
    lse = jnp.log(jnp.sum(ex, axis=1, keepdims=True)) + mx
    o_ref[...] = o - lse


def _head(h, stats, g, be, wl1, bl1, wl2, bl2, d_out):
    n, k = h.shape
    dm = wl1.shape[1]
    pad = 128
    wl2p = jnp.zeros((dm, pad), jnp.float32).at[:, :d_out].set(wl2)
    bl2p = jnp.zeros((1, pad), jnp.float32).at[0, :d_out].set(bl2)
    out = pl.pallas_call(
        functools.partial(_head_body, n=n, d_out=d_out),
        grid=(n // _BN,),
        in_specs=[
            pl.BlockSpec((_BN, k), lambda i: (i, 0)),
            pl.BlockSpec((2, k), lambda i: (0, 0)),
            pl.BlockSpec((1, k), lambda i: (0, 0)),
            pl.BlockSpec((1, k), lambda i: (0, 0)),
            pl.BlockSpec((k, dm), lambda i: (0, 0)),
            pl.BlockSpec((1, dm), lambda i: (0, 0)),
            pl.BlockSpec((dm, pad), lambda i: (0, 0)),
            pl.BlockSpec((1, pad), lambda i: (0, 0)),
        ],
        out_specs=pl.BlockSpec((_BN, pad), lambda i: (i, 0)),
        out_shape=jax.ShapeDtypeStruct((n, pad), jnp.float32),
    )(h, stats, g.reshape(1, k), be.reshape(1, k), wl1, bl1.reshape(1, dm),
      wl2p, bl2p)
    return out[:, :d_out]


def kernel(x, edge_index, edge_weight, W1, b1, g1, be1, W2, b2, g2, be2,
           Wl1, bl1, Wl2, bl2):
    n, d_in = x.shape
    e = edge_weight.shape[0]
    row, col = edge_index[0], edge_index[1]

    degp = _deg_partials(col, edge_weight, n, e)
    dis, xs = _prep(degp, x)

    agg1p = _sc_aggregate(xs, row, col, edge_weight, n, e, 1)
    h1, st1 = _mm_stats(agg1p[:, 0], xs, dis, W1, b1)

    hs2 = _bn_mm(h1, st1, g1, be1, W2, dis)
    agg2p = _sc_aggregate(hs2.reshape(n * 8, 128), row, col, edge_weight,
                          n, e, 8)
    h2, st2 = _comb_stats(agg2p.reshape(16, n, 128), hs2, dis, b2)
    return _head(h2, st2, g2, be2, Wl1, bl1, Wl2, bl2, Wl2.shape[1])
